# baseline (speedup 1.0000x reference)
import jax
import jax.numpy as jnp
import numpy as np
from jax.experimental import pallas as pl
from jax.experimental.pallas import tpu as pltpu
from jax.sharding import Mesh, PartitionSpec as P


def _dot_t(a, b):
    return jax.lax.dot_general(a, b, (((1,), (1,)), ((), ())),
                               preferred_element_type=jnp.float32)


def _moe_body(x_ref, rw_ref, rb_ref, wall_ref, eb_ref, cw_ref, cb_ref, o_ref,
              wall_bf_ref, cw_bf_ref):
    TB = x_ref.shape[0]
    E = rw_ref.shape[0]
    O = eb_ref.shape[1] // E
    RC = 256

    @pl.when(pl.program_id(0) == 0)
    def _cast_weights():
        wall_bf_ref[...] = wall_ref[...].astype(jnp.bfloat16)
        cw_bf_ref[...] = cw_ref[...].astype(jnp.bfloat16)

    logits_full = _dot_t(x_ref[...], rw_ref[...]) + rb_ref[...]

    for c in range(TB // RC):
        rows = slice(c * RC, (c + 1) * RC)
        xb = x_ref[rows, :]
        logits = logits_full[rows, :]
        m = jnp.max(logits, axis=-1, keepdims=True)
        ex = jnp.exp(logits - m)
        s = ex / jnp.sum(ex, axis=-1, keepdims=True)

        eidx = jax.lax.broadcasted_iota(jnp.int32, (RC, E), 1)
        m1 = jnp.max(s, axis=-1, keepdims=True)
        i1 = jnp.min(jnp.where(s == m1, eidx, E), axis=-1, keepdims=True)
        s_m = jnp.where(eidx == i1, -jnp.inf, s)
        m2 = jnp.max(s_m, axis=-1, keepdims=True)
        i2 = jnp.min(jnp.where(s_m == m2, eidx, E), axis=-1, keepdims=True)
        w = jnp.where(eidx == i1, m1, 0.0) + jnp.where(eidx == i2, m2, 0.0)

        xbf = xb.astype(jnp.bfloat16)
        eo = _dot_t(xbf, wall_bf_ref[...]) + eb_ref[...]

        weighted = jnp.zeros((RC, O), dtype=jnp.float32)
        for e in range(E):
            weighted = weighted + eo[:, e * O:(e + 1) * O] * w[:, e:e + 1]

        out = _dot_t(weighted.astype(jnp.bfloat16), cw_bf_ref[...])
        o_ref[rows, :] = out + cb_ref[...]


def _moe_block(xf, router_w, router_b2, wall, eb2, combine_w, cb2):
    T, H = xf.shape
    E = router_w.shape[0]
    O = combine_w.shape[1]
    TB = min(512, T)

    return pl.pallas_call(
        _moe_body,
        grid=(T // TB,),
        in_specs=[
            pl.BlockSpec((TB, H), lambda i: (i, 0)),
            pl.BlockSpec((E, H), lambda i: (0, 0)),
            pl.BlockSpec((1, E), lambda i: (0, 0)),
            pl.BlockSpec((E * O, H), lambda i: (0, 0)),
            pl.BlockSpec((1, E * O), lambda i: (0, 0)),
            pl.BlockSpec((H, O), lambda i: (0, 0)),
            pl.BlockSpec((1, H), lambda i: (0, 0)),
        ],
        out_specs=pl.BlockSpec((TB, H), lambda i: (i, 0)),
        out_shape=jax.ShapeDtypeStruct((T, H), jnp.float32),
        scratch_shapes=[
            pltpu.VMEM((E * O, H), jnp.bfloat16),
            pltpu.VMEM((H, O), jnp.bfloat16),
        ],
    )(xf, router_w, router_b2, wall, eb2, combine_w, cb2)


def kernel(x, router_w, router_b, expert_w, expert_b, combine_w, combine_b):
    B, S, H = x.shape
    E, O = expert_b.shape
    T = B * S

    xf = x.reshape(T, H)
    wall = expert_w.reshape(E * O, H)
    args = (xf, router_w, router_b.reshape(1, E), wall,
            expert_b.reshape(1, E * O), combine_w, combine_b.reshape(1, H))

    ndev = jax.device_count()
    nd = 2 if (ndev >= 2 and T % 1024 == 0) else 1
    if nd > 1:
        mesh = Mesh(np.array(jax.devices()[:nd]), ("d",))
        rep = P(None, None)
        fn = jax.shard_map(
            _moe_block, mesh=mesh,
            in_specs=(P("d", None), rep, rep, rep, rep, rep, rep),
            out_specs=P("d", None), check_vma=False)
        out = fn(*args)
    else:
        out = _moe_block(*args)
    return out.reshape(B, S, H)


# speedup vs baseline: 4.6900x; 4.6900x over previous
import jax
import jax.numpy as jnp
from jax.experimental import pallas as pl
from jax.experimental.pallas import tpu as pltpu


def _dot_t(a, b):
    return jax.lax.dot_general(a, b, (((1,), (1,)), ((), ())),
                               preferred_element_type=jnp.float32)


def _moe_body(x_ref, rw_ref, rb_ref, wall_ref, eb_ref, cw_ref, cb_ref, o_ref,
              wall_bf_ref, cw_bf_ref):
    TB = x_ref.shape[0]
    E = rw_ref.shape[0]
    O = eb_ref.shape[1] // E
    RC = 256

    @pl.when(pl.program_id(0) == 0)
    def _cast_weights():
        wall_bf_ref[...] = wall_ref[...].astype(jnp.bfloat16)
        cw_bf_ref[...] = cw_ref[...].astype(jnp.bfloat16)

    logits_full = _dot_t(x_ref[...], rw_ref[...]) + rb_ref[...]

    for c in range(TB // RC):
        rows = slice(c * RC, (c + 1) * RC)
        xb = x_ref[rows, :]
        logits = logits_full[rows, :]
        m = jnp.max(logits, axis=-1, keepdims=True)
        ex = jnp.exp(logits - m)
        s = ex / jnp.sum(ex, axis=-1, keepdims=True)

        eidx = jax.lax.broadcasted_iota(jnp.int32, (RC, E), 1)
        m1 = jnp.max(s, axis=-1, keepdims=True)
        i1 = jnp.min(jnp.where(s == m1, eidx, E), axis=-1, keepdims=True)
        s_m = jnp.where(eidx == i1, -jnp.inf, s)
        m2 = jnp.max(s_m, axis=-1, keepdims=True)
        i2 = jnp.min(jnp.where(s_m == m2, eidx, E), axis=-1, keepdims=True)
        w = jnp.where(eidx == i1, m1, 0.0) + jnp.where(eidx == i2, m2, 0.0)

        xbf = xb.astype(jnp.bfloat16)
        eo = _dot_t(xbf, wall_bf_ref[...]) + eb_ref[...]

        weighted = jnp.zeros((RC, O), dtype=jnp.float32)
        for e in range(E):
            weighted = weighted + eo[:, e * O:(e + 1) * O] * w[:, e:e + 1]

        out = _dot_t(weighted.astype(jnp.bfloat16), cw_bf_ref[...])
        o_ref[rows, :] = out + cb_ref[...]


def _moe_block(xf, router_w, router_b2, wall, eb2, combine_w, cb2):
    T, H = xf.shape
    E = router_w.shape[0]
    O = combine_w.shape[1]
    TB = min(512, T)

    return pl.pallas_call(
        _moe_body,
        grid=(T // TB,),
        in_specs=[
            pl.BlockSpec((TB, H), lambda i: (i, 0)),
            pl.BlockSpec((E, H), lambda i: (0, 0)),
            pl.BlockSpec((1, E), lambda i: (0, 0)),
            pl.BlockSpec((E * O, H), lambda i: (0, 0)),
            pl.BlockSpec((1, E * O), lambda i: (0, 0)),
            pl.BlockSpec((H, O), lambda i: (0, 0)),
            pl.BlockSpec((1, H), lambda i: (0, 0)),
        ],
        out_specs=pl.BlockSpec((TB, H), lambda i: (i, 0)),
        out_shape=jax.ShapeDtypeStruct((T, H), jnp.float32),
        scratch_shapes=[
            pltpu.VMEM((E * O, H), jnp.bfloat16),
            pltpu.VMEM((H, O), jnp.bfloat16),
        ],
    )(xf, router_w, router_b2, wall, eb2, combine_w, cb2)


def kernel(x, router_w, router_b, expert_w, expert_b, combine_w, combine_b):
    B, S, H = x.shape
    E, O = expert_b.shape
    T = B * S

    xf = x.reshape(T, H)
    wall = expert_w.reshape(E * O, H)
    args = (xf, router_w, router_b.reshape(1, E), wall,
            expert_b.reshape(1, E * O), combine_w, combine_b.reshape(1, H))

    out = _moe_block(*args)
    return out.reshape(B, S, H)
